# baseline (speedup 1.0000x reference)
import dataclasses
import functools

import jax
import jax.numpy as jnp
from jax import lax
from jax.experimental import pallas as pl
from jax.experimental.pallas import tpu as pltpu
from jax.experimental.pallas import tpu_sc as plsc

_HEADS = 4
_KDIM = 512
_HALF = _KDIM // 2
_NKEYS = 512
_SIZE = _NKEYS * _NKEYS
_KNN = 32
_DIM = 1024
_NTOK = 2048
_BAG = _HEADS * _KNN

_NWORKERS = 32
_TPW = _NTOK // _NWORKERS
_GCH = 32
_NCH = _BAG // _GCH


def _bag_body(values_hbm, idx_hbm, w_hbm, out_hbm,
              idx_v, w_v, buf0, buf1, acc, sem0, sem1):
    wid = lax.axis_index("s") * 2 + lax.axis_index("c")
    base = wid * _TPW

    pltpu.sync_copy(idx_hbm.at[pl.ds(base, _TPW)], idx_v)
    pltpu.sync_copy(w_hbm.at[pl.ds(base, _TPW)], w_v)

    bufs = (buf0, buf1)
    sems = (sem0, sem1)

    def start(t, c):
        return pltpu.async_copy(
            values_hbm.at[idx_v.at[t, pl.ds(c * _GCH, _GCH)]],
            bufs[c % 2], sems[c % 2])

    @pl.loop(0, _TPW)
    def _token(t):
        copies = [start(t, 0), start(t, 1)]
        for c in range(_NCH):
            wregs = [
                plsc.load_gather(
                    w_v, [jnp.full((16,), t, jnp.int32),
                          jnp.full((16,), c * _GCH + r, jnp.int32)])
                for r in range(_GCH)
            ]
            copies[c % 2].wait()
            buf = bufs[c % 2]
            first = c == 0

            @pl.loop(0, _DIM, step=16)
            def _col(ci):
                sl = pl.ds(ci, 16)
                parts = [wregs[p] * buf[p, sl] for p in range(4)]
                for r in range(4, _GCH):
                    parts[r % 4] = parts[r % 4] + wregs[r] * buf[r, sl]
                s = (parts[0] + parts[1]) + (parts[2] + parts[3])
                if first:
                    acc[sl] = s
                else:
                    plsc.addupdate(acc.at[sl], s)

            if c + 2 < _NCH:
                copies[c % 2] = start(t, c + 2)

        pltpu.sync_copy(acc, out_hbm.at[base + t])


def _sc_bag(values, idx, w):
    mesh = plsc.VectorSubcoreMesh(core_axis_name="c", subcore_axis_name="s")
    cp = pltpu.CompilerParams()
    if "needs_layout_passes" in pltpu.CompilerParams.__dataclass_fields__:
        cp = dataclasses.replace(cp, needs_layout_passes=False)
    kern = functools.partial(
        pl.kernel,
        compiler_params=cp,
        out_type=jax.ShapeDtypeStruct((_NTOK, _DIM), jnp.float32),
        mesh=mesh,
        scratch_types=[
            pltpu.VMEM((_TPW, _BAG), jnp.int32),
            pltpu.VMEM((_TPW, _BAG), jnp.float32),
            pltpu.VMEM((_GCH, _DIM), jnp.float32),
            pltpu.VMEM((_GCH, _DIM), jnp.float32),
            pltpu.VMEM((_DIM,), jnp.float32),
            pltpu.SemaphoreType.DMA,
            pltpu.SemaphoreType.DMA,
        ],
    )(_bag_body)
    return kern(values, idx, w)


def kernel(x, W_q, b_q, keys_p, values, W_swilu, b_swilu, W_vproj, b_vproj):
    bs = x.shape[0]
    q = x @ W_q.T + b_q
    q = q.reshape(bs, _HEADS, _KDIM)
    q1 = q[..., :_HALF]
    q2 = q[..., _HALF:]
    keys_r = keys_p.reshape(_HEADS, 2, _NKEYS, _HALF)
    s1 = jnp.einsum('bhd,hnd->bhn', q1, keys_r[:, 0])
    s2 = jnp.einsum('bhd,hnd->bhn', q2, keys_r[:, 1])
    v1, i1 = jax.lax.top_k(s1, _KNN)
    v2, i2 = jax.lax.top_k(s2, _KNN)
    all_s = (v1[..., :, None] + v2[..., None, :]).reshape(bs, _HEADS, _KNN * _KNN)
    all_i = (i1[..., :, None] * _NKEYS + i2[..., None, :]).reshape(bs, _HEADS, _KNN * _KNN)
    best_s, best_pos = jax.lax.top_k(all_s, _KNN)
    best_i = jnp.take_along_axis(all_i, best_pos, axis=-1)
    w = jax.nn.softmax(best_s, axis=-1)

    idx2d = best_i.reshape(bs, _BAG).astype(jnp.int32)
    w2d = w.reshape(bs, _BAG)
    bag = _sc_bag(values, idx2d, w2d)

    out = bag * jax.nn.silu(x @ W_swilu.T + b_swilu)
    out = out @ W_vproj.T + b_vproj
    return out


# speedup vs baseline: 1.4252x; 1.4252x over previous
import dataclasses
import functools

import jax
import jax.numpy as jnp
from jax import lax
from jax.experimental import pallas as pl
from jax.experimental.pallas import tpu as pltpu
from jax.experimental.pallas import tpu_sc as plsc

_HEADS = 4
_KDIM = 512
_HALF = _KDIM // 2
_NKEYS = 512
_SIZE = _NKEYS * _NKEYS
_KNN = 32
_DIM = 1024
_NTOK = 2048
_BAG = _HEADS * _KNN

_NWORKERS = 32
_TPW = _NTOK // _NWORKERS
_GCH = 32
_NCH = _BAG // _GCH


def _bag_body(values_hbm, idx_hbm, w_hbm, out_hbm,
              idx_v, w_v, buf0, buf1, acc, sem0, sem1):
    wid = lax.axis_index("s") * 2 + lax.axis_index("c")
    base = wid * _TPW

    pltpu.sync_copy(idx_hbm.at[pl.ds(base, _TPW)], idx_v)
    pltpu.sync_copy(w_hbm.at[pl.ds(base, _TPW)], w_v)

    bufs = (buf0, buf1)
    sems = (sem0, sem1)

    def start(t, c):
        return pltpu.async_copy(
            values_hbm.at[idx_v.at[t, pl.ds(c * _GCH, _GCH)]],
            bufs[c % 2], sems[c % 2])

    @pl.loop(0, _TPW)
    def _token(t):
        copies = [start(t, 0), start(t, 1)]
        for c in range(_NCH):
            wregs = [
                plsc.load_gather(
                    w_v, [jnp.full((16,), t, jnp.int32),
                          jnp.full((16,), c * _GCH + r, jnp.int32)])
                for r in range(_GCH)
            ]
            copies[c % 2].wait()
            buf = bufs[c % 2]
            first = c == 0

            @pl.loop(0, _DIM, step=16)
            def _col(ci):
                sl = pl.ds(ci, 16)
                parts = [wregs[p] * buf[p, sl] for p in range(4)]
                for r in range(4, _GCH):
                    parts[r % 4] = parts[r % 4] + wregs[r] * buf[r, sl]
                s = (parts[0] + parts[1]) + (parts[2] + parts[3])
                if first:
                    acc[sl] = s
                else:
                    plsc.addupdate(acc.at[sl], s)

            if c + 2 < _NCH:
                copies[c % 2] = start(t, c + 2)

        pltpu.sync_copy(acc, out_hbm.at[base + t])


def _sc_bag(values, idx, w):
    mesh = plsc.VectorSubcoreMesh(core_axis_name="c", subcore_axis_name="s")
    cp = pltpu.CompilerParams()
    if "needs_layout_passes" in pltpu.CompilerParams.__dataclass_fields__:
        cp = dataclasses.replace(cp, needs_layout_passes=False)
    kern = functools.partial(
        pl.kernel,
        compiler_params=cp,
        out_type=jax.ShapeDtypeStruct((_NTOK, _DIM), jnp.float32),
        mesh=mesh,
        scratch_types=[
            pltpu.VMEM((_TPW, _BAG), jnp.int32),
            pltpu.VMEM((_TPW, _BAG), jnp.float32),
            pltpu.VMEM((_GCH, _DIM), jnp.float32),
            pltpu.VMEM((_GCH, _DIM), jnp.float32),
            pltpu.VMEM((_DIM,), jnp.float32),
            pltpu.SemaphoreType.DMA,
            pltpu.SemaphoreType.DMA,
        ],
    )(_bag_body)
    return kern(values, idx, w)


_BT = 256
_NJ = [32 // (i + 1) for i in range(_KNN)]
_NCAND = sum(_NJ)


def _extract_topk(cur, pay, k_out):
    rows = cur.shape[0]
    iota = jax.lax.broadcasted_iota(jnp.int32, (rows, k_out), 1)
    big = jnp.int32(1 << 30)

    def body(k, carry):
        thr, tpay, vout, pout = carry
        ext = (cur > thr) | ((cur == thr) & (pay <= tpay))
        masked = jnp.where(ext, -jnp.inf, cur)
        m = jnp.max(masked, axis=1, keepdims=True)
        psel = jnp.min(jnp.where(masked == m, pay, big), axis=1,
                       keepdims=True)
        vout = jnp.where(iota == k, m, vout)
        pout = jnp.where(iota == k, psel, pout)
        return (m, psel, vout, pout)

    thr0 = jnp.full((rows, 1), jnp.inf, jnp.float32)
    tpay0 = jnp.full((rows, 1), big, jnp.int32)
    vout0 = jnp.zeros((rows, k_out), jnp.float32)
    pout0 = jnp.zeros((rows, k_out), jnp.int32)
    _, _, vout, pout = jax.lax.fori_loop(
        0, k_out, body, (thr0, tpay0, vout0, pout0))
    return vout, pout


def _k1_body(x_ref, wqt_ref, bq_ref, kt_ref, wst_ref, bs_ref,
             w_ref, idx_ref, gate_ref):
    xb = x_ref[...]
    q = jnp.dot(xb, wqt_ref[...], preferred_element_type=jnp.float32) + bq_ref[...]

    lane9 = jax.lax.broadcasted_iota(jnp.int32, (_BT, _NKEYS), 1)
    w_heads, idx_heads = [], []
    for h in range(_HEADS):
        q1 = q[:, h * _KDIM:h * _KDIM + _HALF]
        q2 = q[:, h * _KDIM + _HALF:(h + 1) * _KDIM]
        s1 = jnp.dot(q1, kt_ref[h, 0], preferred_element_type=jnp.float32)
        s2 = jnp.dot(q2, kt_ref[h, 1], preferred_element_type=jnp.float32)
        v1, i1 = _extract_topk(s1, lane9, _KNN)
        v2, i2 = _extract_topk(s2, lane9, _KNN)

        pieces, codes = [], []
        for i in range(_KNN):
            n = _NJ[i]
            pieces.append(v1[:, i:i + 1] + v2[:, :n])
            codes.append(i * _KNN +
                         jax.lax.broadcasted_iota(jnp.int32, (_BT, n), 1))
        cand = jnp.concatenate(pieces, axis=1)
        code_row = jnp.concatenate(codes, axis=1)
        bv, code = _extract_topk(cand, code_row, _KNN)
        k1 = code >> 5
        k2 = code & 31

        m0 = bv[:, 0:1]
        e = jnp.exp(bv - m0)
        wgt = e / jnp.sum(e, axis=1, keepdims=True)

        i1sel = jnp.zeros((_BT, _KNN), jnp.int32)
        i2sel = jnp.zeros((_BT, _KNN), jnp.int32)
        for j in range(_KNN):
            i1sel = jnp.where(k1 == j, i1[:, j:j + 1], i1sel)
            i2sel = jnp.where(k2 == j, i2[:, j:j + 1], i2sel)
        w_heads.append(wgt)
        idx_heads.append(i1sel * _NKEYS + i2sel)

    w_ref[...] = jnp.concatenate(w_heads, axis=1)
    idx_ref[...] = jnp.concatenate(idx_heads, axis=1)

    z = jnp.dot(xb, wst_ref[...], preferred_element_type=jnp.float32) + bs_ref[...]
    gate_ref[...] = z / (1.0 + jnp.exp(-z))


def _tc_search_and_gate(x, W_qT, b_q2, keysT, W_swiluT, b_s2):
    grid = (_NTOK // _BT,)
    return pl.pallas_call(
        _k1_body,
        grid=grid,
        in_specs=[
            pl.BlockSpec((_BT, _DIM), lambda i: (i, 0)),
            pl.BlockSpec((_DIM, _HEADS * _KDIM), lambda i: (0, 0)),
            pl.BlockSpec((1, _HEADS * _KDIM), lambda i: (0, 0)),
            pl.BlockSpec((_HEADS, 2, _HALF, _NKEYS), lambda i: (0, 0, 0, 0)),
            pl.BlockSpec((_DIM, _DIM), lambda i: (0, 0)),
            pl.BlockSpec((1, _DIM), lambda i: (0, 0)),
        ],
        out_specs=[
            pl.BlockSpec((_BT, _BAG), lambda i: (i, 0)),
            pl.BlockSpec((_BT, _BAG), lambda i: (i, 0)),
            pl.BlockSpec((_BT, _DIM), lambda i: (i, 0)),
        ],
        out_shape=[
            jax.ShapeDtypeStruct((_NTOK, _BAG), jnp.float32),
            jax.ShapeDtypeStruct((_NTOK, _BAG), jnp.int32),
            jax.ShapeDtypeStruct((_NTOK, _DIM), jnp.float32),
        ],
    )(x, W_qT, b_q2, keysT, W_swiluT, b_s2)


def _k3_body(bag_ref, gate_ref, wvt_ref, bv_ref, o_ref):
    o_ref[...] = jnp.dot(bag_ref[...] * gate_ref[...], wvt_ref[...],
                         preferred_element_type=jnp.float32) + bv_ref[...]


def _tc_vproj(bag, gate, W_vprojT, b_v2):
    return pl.pallas_call(
        _k3_body,
        grid=(_NTOK // _BT,),
        in_specs=[
            pl.BlockSpec((_BT, _DIM), lambda i: (i, 0)),
            pl.BlockSpec((_BT, _DIM), lambda i: (i, 0)),
            pl.BlockSpec((_DIM, _DIM), lambda i: (0, 0)),
            pl.BlockSpec((1, _DIM), lambda i: (0, 0)),
        ],
        out_specs=pl.BlockSpec((_BT, _DIM), lambda i: (i, 0)),
        out_shape=jax.ShapeDtypeStruct((_NTOK, _DIM), jnp.float32),
    )(bag, gate, W_vprojT, b_v2)


def kernel(x, W_q, b_q, keys_p, values, W_swilu, b_swilu, W_vproj, b_vproj):
    keysT = keys_p.reshape(_HEADS, 2, _NKEYS, _HALF).transpose(0, 1, 3, 2)
    w2d, idx2d, gate = _tc_search_and_gate(
        x, W_q.T, b_q[None, :], keysT, W_swilu.T, b_swilu[None, :])
    bag = _sc_bag(values, idx2d, w2d)
    return _tc_vproj(bag, gate, W_vproj.T, b_vproj[None, :])


# speedup vs baseline: 1.5419x; 1.0819x over previous
import dataclasses
import functools

import jax
import jax.numpy as jnp
from jax import lax
from jax.experimental import pallas as pl
from jax.experimental.pallas import tpu as pltpu
from jax.experimental.pallas import tpu_sc as plsc

_HEADS = 4
_KDIM = 512
_HALF = _KDIM // 2
_NKEYS = 512
_SIZE = _NKEYS * _NKEYS
_KNN = 32
_DIM = 1024
_NTOK = 2048
_BAG = _HEADS * _KNN

_NWORKERS = 32
_TPW = _NTOK // _NWORKERS
_GCH = 32
_NCH = _BAG // _GCH


def _bag_body(values_hbm, idx_hbm, w_hbm, out_hbm,
              idx_v, w_v, buf0, buf1, acc, sem0, sem1):
    wid = lax.axis_index("s") * 2 + lax.axis_index("c")
    base = wid * _TPW

    pltpu.sync_copy(idx_hbm.at[pl.ds(base, _TPW)], idx_v)
    pltpu.sync_copy(w_hbm.at[pl.ds(base, _TPW)], w_v)

    bufs = (buf0, buf1)
    sems = (sem0, sem1)

    def start(t, c):
        return pltpu.async_copy(
            values_hbm.at[idx_v.at[t, pl.ds(c * _GCH, _GCH)]],
            bufs[c % 2], sems[c % 2])

    @pl.loop(0, _TPW)
    def _token(t):
        copies = [start(t, 0), start(t, 1)]
        for c in range(_NCH):
            wregs = [
                plsc.load_gather(
                    w_v, [jnp.full((16,), t, jnp.int32),
                          jnp.full((16,), c * _GCH + r, jnp.int32)])
                for r in range(_GCH)
            ]
            copies[c % 2].wait()
            buf = bufs[c % 2]
            first = c == 0

            @pl.loop(0, _DIM, step=16)
            def _col(ci):
                sl = pl.ds(ci, 16)
                parts = [wregs[p] * buf[p, sl] for p in range(4)]
                for r in range(4, _GCH):
                    parts[r % 4] = parts[r % 4] + wregs[r] * buf[r, sl]
                s = (parts[0] + parts[1]) + (parts[2] + parts[3])
                if first:
                    acc[sl] = s
                else:
                    plsc.addupdate(acc.at[sl], s)

            if c + 2 < _NCH:
                copies[c % 2] = start(t, c + 2)

        pltpu.sync_copy(acc, out_hbm.at[base + t])


def _sc_bag(values, idx, w):
    mesh = plsc.VectorSubcoreMesh(core_axis_name="c", subcore_axis_name="s")
    cp = pltpu.CompilerParams()
    if "needs_layout_passes" in pltpu.CompilerParams.__dataclass_fields__:
        cp = dataclasses.replace(cp, needs_layout_passes=False)
    kern = functools.partial(
        pl.kernel,
        compiler_params=cp,
        out_type=jax.ShapeDtypeStruct((_NTOK, _DIM), jnp.float32),
        mesh=mesh,
        scratch_types=[
            pltpu.VMEM((_TPW, _BAG), jnp.int32),
            pltpu.VMEM((_TPW, _BAG), jnp.float32),
            pltpu.VMEM((_GCH, _DIM), jnp.float32),
            pltpu.VMEM((_GCH, _DIM), jnp.float32),
            pltpu.VMEM((_DIM,), jnp.float32),
            pltpu.SemaphoreType.DMA,
            pltpu.SemaphoreType.DMA,
        ],
    )(_bag_body)
    return kern(values, idx, w)


_BT = 256
_NJ = [32 // (i + 1) for i in range(_KNN)]
_NCAND = sum(_NJ)


def _extract_topk(msk_ref, src, pay, k_out):
    rows, n = src.shape
    sl = (slice(None), slice(0, n))
    msk_ref[sl] = src
    iota = jax.lax.broadcasted_iota(jnp.int32, (rows, k_out), 1)
    big = jnp.int32(1 << 30)

    def body(k, carry):
        vout, pout = carry
        cur = msk_ref[sl]
        m = jnp.max(cur, axis=1, keepdims=True)
        match = cur == m
        psel = jnp.min(jnp.where(match, pay, big), axis=1, keepdims=True)
        msk_ref[sl] = jnp.where(match & (pay == psel), -jnp.inf, cur)
        vout = jnp.where(iota == k, m, vout)
        pout = jnp.where(iota == k, psel, pout)
        return (vout, pout)

    vout0 = jnp.zeros((rows, k_out), jnp.float32)
    pout0 = jnp.zeros((rows, k_out), jnp.int32)
    vout, pout = jax.lax.fori_loop(0, k_out, body, (vout0, pout0))
    return vout, pout


def _k1_body(x_ref, wqt_ref, bq_ref, kt_ref, wst_ref, bs_ref,
             w_ref, idx_ref, gate_ref, msk_ref):
    xb = x_ref[...]
    q = jnp.dot(xb, wqt_ref[...], preferred_element_type=jnp.float32) + bq_ref[...]

    lane9 = jax.lax.broadcasted_iota(jnp.int32, (_BT, _NKEYS), 1)
    w_heads, idx_heads = [], []
    for h in range(_HEADS):
        q1 = q[:, h * _KDIM:h * _KDIM + _HALF]
        q2 = q[:, h * _KDIM + _HALF:(h + 1) * _KDIM]
        s1 = jnp.dot(q1, kt_ref[h, 0], preferred_element_type=jnp.float32)
        s2 = jnp.dot(q2, kt_ref[h, 1], preferred_element_type=jnp.float32)
        v1, i1 = _extract_topk(msk_ref, s1, lane9, _KNN)
        v2, i2 = _extract_topk(msk_ref, s2, lane9, _KNN)

        pieces, codes = [], []
        for i in range(_KNN):
            n = _NJ[i]
            pieces.append(v1[:, i:i + 1] + v2[:, :n])
            codes.append(i * _KNN +
                         jax.lax.broadcasted_iota(jnp.int32, (_BT, n), 1))
        cand = jnp.concatenate(pieces, axis=1)
        code_row = jnp.concatenate(codes, axis=1)
        bv, code = _extract_topk(msk_ref, cand, code_row, _KNN)
        k1 = code >> 5
        k2 = code & 31

        m0 = bv[:, 0:1]
        e = jnp.exp(bv - m0)
        wgt = e / jnp.sum(e, axis=1, keepdims=True)

        i1sel = jnp.zeros((_BT, _KNN), jnp.int32)
        i2sel = jnp.zeros((_BT, _KNN), jnp.int32)
        for j in range(_KNN):
            i1sel = jnp.where(k1 == j, i1[:, j:j + 1], i1sel)
            i2sel = jnp.where(k2 == j, i2[:, j:j + 1], i2sel)
        w_heads.append(wgt)
        idx_heads.append(i1sel * _NKEYS + i2sel)

    w_ref[...] = jnp.concatenate(w_heads, axis=1)
    idx_ref[...] = jnp.concatenate(idx_heads, axis=1)

    z = jnp.dot(xb, wst_ref[...], preferred_element_type=jnp.float32) + bs_ref[...]
    gate_ref[...] = z / (1.0 + jnp.exp(-z))


def _tc_search_and_gate(x, W_qT, b_q2, keysT, W_swiluT, b_s2):
    grid = (_NTOK // _BT,)
    return pl.pallas_call(
        _k1_body,
        grid=grid,
        in_specs=[
            pl.BlockSpec((_BT, _DIM), lambda i: (i, 0)),
            pl.BlockSpec((_DIM, _HEADS * _KDIM), lambda i: (0, 0)),
            pl.BlockSpec((1, _HEADS * _KDIM), lambda i: (0, 0)),
            pl.BlockSpec((_HEADS, 2, _HALF, _NKEYS), lambda i: (0, 0, 0, 0)),
            pl.BlockSpec((_DIM, _DIM), lambda i: (0, 0)),
            pl.BlockSpec((1, _DIM), lambda i: (0, 0)),
        ],
        out_specs=[
            pl.BlockSpec((_BT, _BAG), lambda i: (i, 0)),
            pl.BlockSpec((_BT, _BAG), lambda i: (i, 0)),
            pl.BlockSpec((_BT, _DIM), lambda i: (i, 0)),
        ],
        out_shape=[
            jax.ShapeDtypeStruct((_NTOK, _BAG), jnp.float32),
            jax.ShapeDtypeStruct((_NTOK, _BAG), jnp.int32),
            jax.ShapeDtypeStruct((_NTOK, _DIM), jnp.float32),
        ],
        scratch_shapes=[pltpu.VMEM((_BT, _NKEYS), jnp.float32)],
    )(x, W_qT, b_q2, keysT, W_swiluT, b_s2)


def _k3_body(bag_ref, gate_ref, wvt_ref, bv_ref, o_ref):
    o_ref[...] = jnp.dot(bag_ref[...] * gate_ref[...], wvt_ref[...],
                         preferred_element_type=jnp.float32) + bv_ref[...]


def _tc_vproj(bag, gate, W_vprojT, b_v2):
    return pl.pallas_call(
        _k3_body,
        grid=(_NTOK // _BT,),
        in_specs=[
            pl.BlockSpec((_BT, _DIM), lambda i: (i, 0)),
            pl.BlockSpec((_BT, _DIM), lambda i: (i, 0)),
            pl.BlockSpec((_DIM, _DIM), lambda i: (0, 0)),
            pl.BlockSpec((1, _DIM), lambda i: (0, 0)),
        ],
        out_specs=pl.BlockSpec((_BT, _DIM), lambda i: (i, 0)),
        out_shape=jax.ShapeDtypeStruct((_NTOK, _DIM), jnp.float32),
    )(bag, gate, W_vprojT, b_v2)


def kernel(x, W_q, b_q, keys_p, values, W_swilu, b_swilu, W_vproj, b_vproj):
    keysT = keys_p.reshape(_HEADS, 2, _NKEYS, _HALF).transpose(0, 1, 3, 2)
    w2d, idx2d, gate = _tc_search_and_gate(
        x, W_q.T, b_q[None, :], keysT, W_swilu.T, b_swilu[None, :])
    bag = _sc_bag(values, idx2d, w2d)
    return _tc_vproj(bag, gate, W_vproj.T, b_vproj[None, :])


# speedup vs baseline: 1.9688x; 1.2769x over previous
import dataclasses
import functools

import jax
import jax.numpy as jnp
from jax import lax
from jax.experimental import pallas as pl
from jax.experimental.pallas import tpu as pltpu
from jax.experimental.pallas import tpu_sc as plsc

_HEADS = 4
_KDIM = 512
_HALF = _KDIM // 2
_NKEYS = 512
_SIZE = _NKEYS * _NKEYS
_KNN = 32
_DIM = 1024
_NTOK = 2048
_BAG = _HEADS * _KNN

_NWORKERS = 32
_TPW = _NTOK // _NWORKERS
_GCH = 32
_NCH = _BAG // _GCH


def _bag_body(tpw, values_hbm, idx_hbm, w_hbm, out_hbm,
              idx_v, w_v, buf0, buf1, acc, sem0, sem1):
    wid = lax.axis_index("s") * 2 + lax.axis_index("c")
    base = wid * tpw

    pltpu.sync_copy(idx_hbm.at[pl.ds(base, tpw)], idx_v)
    pltpu.sync_copy(w_hbm.at[pl.ds(base, tpw)], w_v)

    bufs = (buf0, buf1)
    sems = (sem0, sem1)

    def start(t, c):
        return pltpu.async_copy(
            values_hbm.at[idx_v.at[t, pl.ds(c * _GCH, _GCH)]],
            bufs[c % 2], sems[c % 2])

    @pl.loop(0, tpw)
    def _token(t):
        copies = [start(t, 0), start(t, 1)]
        for c in range(_NCH):
            wregs = [
                plsc.load_gather(
                    w_v, [jnp.full((16,), t, jnp.int32),
                          jnp.full((16,), c * _GCH + r, jnp.int32)])
                for r in range(_GCH)
            ]
            copies[c % 2].wait()
            buf = bufs[c % 2]
            first = c == 0

            @pl.loop(0, _DIM, step=16)
            def _col(ci):
                sl = pl.ds(ci, 16)
                parts = [wregs[p] * buf[p, sl] for p in range(4)]
                for r in range(4, _GCH):
                    parts[r % 4] = parts[r % 4] + wregs[r] * buf[r, sl]
                s = (parts[0] + parts[1]) + (parts[2] + parts[3])
                if first:
                    acc[sl] = s
                else:
                    plsc.addupdate(acc.at[sl], s)

            if c + 2 < _NCH:
                copies[c % 2] = start(t, c + 2)

        pltpu.sync_copy(acc, out_hbm.at[base + t])


def _sc_bag(values, idx, w):
    ntok = idx.shape[0]
    tpw = ntok // _NWORKERS
    mesh = plsc.VectorSubcoreMesh(core_axis_name="c", subcore_axis_name="s")
    cp = pltpu.CompilerParams()
    if "needs_layout_passes" in pltpu.CompilerParams.__dataclass_fields__:
        cp = dataclasses.replace(cp, needs_layout_passes=False)
    kern = functools.partial(
        pl.kernel,
        compiler_params=cp,
        out_type=jax.ShapeDtypeStruct((ntok, _DIM), jnp.float32),
        mesh=mesh,
        scratch_types=[
            pltpu.VMEM((tpw, _BAG), jnp.int32),
            pltpu.VMEM((tpw, _BAG), jnp.float32),
            pltpu.VMEM((_GCH, _DIM), jnp.float32),
            pltpu.VMEM((_GCH, _DIM), jnp.float32),
            pltpu.VMEM((_DIM,), jnp.float32),
            pltpu.SemaphoreType.DMA,
            pltpu.SemaphoreType.DMA,
        ],
    )(functools.partial(_bag_body, tpw))
    return kern(values, idx, w)


_BT = 256
_NJ = [32 // (i + 1) for i in range(_KNN)]
_NCAND = sum(_NJ)


def _extract_topk(msk_ref, src, pay, k_out):
    rows, n = src.shape
    sl = (slice(None), slice(0, n))
    msk_ref[sl] = src
    iota = jax.lax.broadcasted_iota(jnp.int32, (rows, k_out), 1)
    big = jnp.int32(1 << 30)

    def body(k, carry):
        vout, pout = carry
        cur = msk_ref[sl]
        m = jnp.max(cur, axis=1, keepdims=True)
        match = cur == m
        psel = jnp.min(jnp.where(match, pay, big), axis=1, keepdims=True)
        msk_ref[sl] = jnp.where(match & (pay == psel), -jnp.inf, cur)
        vout = jnp.where(iota == k, m, vout)
        pout = jnp.where(iota == k, psel, pout)
        return (vout, pout)

    vout0 = jnp.zeros((rows, k_out), jnp.float32)
    pout0 = jnp.zeros((rows, k_out), jnp.int32)
    vout, pout = jax.lax.fori_loop(0, k_out, body, (vout0, pout0))
    return vout, pout


def _k1_body(x_ref, wqt_ref, bq_ref, kt_ref, wst_ref, bs_ref,
             w_ref, idx_ref, gate_ref, msk_ref):
    xb = x_ref[...]
    q = jnp.dot(xb, wqt_ref[...], preferred_element_type=jnp.float32) + bq_ref[...]

    lane9 = jax.lax.broadcasted_iota(jnp.int32, (_BT, _NKEYS), 1)
    w_heads, idx_heads = [], []
    for h in range(_HEADS):
        q1 = q[:, h * _KDIM:h * _KDIM + _HALF]
        q2 = q[:, h * _KDIM + _HALF:(h + 1) * _KDIM]
        s1 = jnp.dot(q1, kt_ref[h, 0], preferred_element_type=jnp.float32)
        s2 = jnp.dot(q2, kt_ref[h, 1], preferred_element_type=jnp.float32)
        v1, i1 = _extract_topk(msk_ref, s1, lane9, _KNN)
        v2, i2 = _extract_topk(msk_ref, s2, lane9, _KNN)

        pieces, codes = [], []
        for i in range(_KNN):
            n = _NJ[i]
            pieces.append(v1[:, i:i + 1] + v2[:, :n])
            codes.append(i * _KNN +
                         jax.lax.broadcasted_iota(jnp.int32, (_BT, n), 1))
        cand = jnp.concatenate(pieces, axis=1)
        code_row = jnp.concatenate(codes, axis=1)
        bv, code = _extract_topk(msk_ref, cand, code_row, _KNN)
        k1 = code >> 5
        k2 = code & 31

        m0 = bv[:, 0:1]
        e = jnp.exp(bv - m0)
        wgt = e / jnp.sum(e, axis=1, keepdims=True)

        i1sel = jnp.zeros((_BT, _KNN), jnp.int32)
        i2sel = jnp.zeros((_BT, _KNN), jnp.int32)
        for j in range(_KNN):
            i1sel = jnp.where(k1 == j, i1[:, j:j + 1], i1sel)
            i2sel = jnp.where(k2 == j, i2[:, j:j + 1], i2sel)
        w_heads.append(wgt)
        idx_heads.append(i1sel * _NKEYS + i2sel)

    w_ref[...] = jnp.concatenate(w_heads, axis=1)
    idx_ref[...] = jnp.concatenate(idx_heads, axis=1)

    z = jnp.dot(xb, wst_ref[...], preferred_element_type=jnp.float32) + bs_ref[...]
    gate_ref[...] = z / (1.0 + jnp.exp(-z))


def _tc_search_and_gate(x, W_qT, b_q2, keysT, W_swiluT, b_s2):
    ntok = x.shape[0]
    grid = (ntok // _BT,)
    return pl.pallas_call(
        _k1_body,
        grid=grid,
        in_specs=[
            pl.BlockSpec((_BT, _DIM), lambda i: (i, 0)),
            pl.BlockSpec((_DIM, _HEADS * _KDIM), lambda i: (0, 0)),
            pl.BlockSpec((1, _HEADS * _KDIM), lambda i: (0, 0)),
            pl.BlockSpec((_HEADS, 2, _HALF, _NKEYS), lambda i: (0, 0, 0, 0)),
            pl.BlockSpec((_DIM, _DIM), lambda i: (0, 0)),
            pl.BlockSpec((1, _DIM), lambda i: (0, 0)),
        ],
        out_specs=[
            pl.BlockSpec((_BT, _BAG), lambda i: (i, 0)),
            pl.BlockSpec((_BT, _BAG), lambda i: (i, 0)),
            pl.BlockSpec((_BT, _DIM), lambda i: (i, 0)),
        ],
        out_shape=[
            jax.ShapeDtypeStruct((ntok, _BAG), jnp.float32),
            jax.ShapeDtypeStruct((ntok, _BAG), jnp.int32),
            jax.ShapeDtypeStruct((ntok, _DIM), jnp.float32),
        ],
        scratch_shapes=[pltpu.VMEM((_BT, _NKEYS), jnp.float32)],
    )(x, W_qT, b_q2, keysT, W_swiluT, b_s2)


def _k3_body(bag_ref, gate_ref, wvt_ref, bv_ref, o_ref):
    o_ref[...] = jnp.dot(bag_ref[...] * gate_ref[...], wvt_ref[...],
                         preferred_element_type=jnp.float32) + bv_ref[...]


def _tc_vproj(bag, gate, W_vprojT, b_v2):
    return pl.pallas_call(
        _k3_body,
        grid=(bag.shape[0] // _BT,),
        in_specs=[
            pl.BlockSpec((_BT, _DIM), lambda i: (i, 0)),
            pl.BlockSpec((_BT, _DIM), lambda i: (i, 0)),
            pl.BlockSpec((_DIM, _DIM), lambda i: (0, 0)),
            pl.BlockSpec((1, _DIM), lambda i: (0, 0)),
        ],
        out_specs=pl.BlockSpec((_BT, _DIM), lambda i: (i, 0)),
        out_shape=jax.ShapeDtypeStruct((bag.shape[0], _DIM), jnp.float32),
    )(bag, gate, W_vprojT, b_v2)


_NCHUNK = 4


def kernel(x, W_q, b_q, keys_p, values, W_swilu, b_swilu, W_vproj, b_vproj):
    keysT = keys_p.reshape(_HEADS, 2, _NKEYS, _HALF).transpose(0, 1, 3, 2)
    W_qT, W_swiluT, W_vprojT = W_q.T, W_swilu.T, W_vproj.T
    bq2, bs2, bv2 = b_q[None, :], b_swilu[None, :], b_vproj[None, :]
    step = _NTOK // _NCHUNK
    outs = []
    for c in range(_NCHUNK):
        xc = jax.lax.slice_in_dim(x, c * step, (c + 1) * step, axis=0)
        w2d, idx2d, gate = _tc_search_and_gate(
            xc, W_qT, bq2, keysT, W_swiluT, bs2)
        bag = _sc_bag(values, idx2d, w2d)
        outs.append(_tc_vproj(bag, gate, W_vprojT, bv2))
    return jnp.concatenate(outs, axis=0)


# speedup vs baseline: 2.1257x; 1.0797x over previous
import dataclasses
import functools

import jax
import jax.numpy as jnp
from jax import lax
from jax.experimental import pallas as pl
from jax.experimental.pallas import tpu as pltpu
from jax.experimental.pallas import tpu_sc as plsc

_HEADS = 4
_KDIM = 512
_HALF = _KDIM // 2
_NKEYS = 512
_SIZE = _NKEYS * _NKEYS
_KNN = 32
_DIM = 1024
_NTOK = 2048
_BAG = _HEADS * _KNN

_NWORKERS = 32
_TPW = _NTOK // _NWORKERS
_GCH = 32
_NCH = _BAG // _GCH


def _bag_body(tpw, values_hbm, idx_hbm, w_hbm, out_hbm,
              idx_v, w_v, buf0, buf1, acc, sem0, sem1):
    wid = lax.axis_index("s") * 2 + lax.axis_index("c")
    base = wid * tpw

    pltpu.sync_copy(idx_hbm.at[pl.ds(base, tpw)], idx_v)
    pltpu.sync_copy(w_hbm.at[pl.ds(base, tpw)], w_v)

    bufs = (buf0, buf1)
    sems = (sem0, sem1)

    def start(t, c):
        return pltpu.async_copy(
            values_hbm.at[idx_v.at[t, pl.ds(c * _GCH, _GCH)]],
            bufs[c % 2], sems[c % 2])

    @pl.loop(0, tpw)
    def _token(t):
        copies = [start(t, 0), start(t, 1)]
        for c in range(_NCH):
            wregs = [
                plsc.load_gather(
                    w_v, [jnp.full((16,), t, jnp.int32),
                          jnp.full((16,), c * _GCH + r, jnp.int32)])
                for r in range(_GCH)
            ]
            copies[c % 2].wait()
            buf = bufs[c % 2]
            first = c == 0

            @pl.loop(0, _DIM, step=16)
            def _col(ci):
                sl = pl.ds(ci, 16)
                parts = [wregs[p] * buf[p, sl] for p in range(4)]
                for r in range(4, _GCH):
                    parts[r % 4] = parts[r % 4] + wregs[r] * buf[r, sl]
                s = (parts[0] + parts[1]) + (parts[2] + parts[3])
                if first:
                    acc[sl] = s
                else:
                    plsc.addupdate(acc.at[sl], s)

            if c + 2 < _NCH:
                copies[c % 2] = start(t, c + 2)

        pltpu.sync_copy(acc, out_hbm.at[base + t])


def _sc_bag(values, idx, w):
    ntok = idx.shape[0]
    tpw = ntok // _NWORKERS
    mesh = plsc.VectorSubcoreMesh(core_axis_name="c", subcore_axis_name="s")
    cp = pltpu.CompilerParams()
    if "needs_layout_passes" in pltpu.CompilerParams.__dataclass_fields__:
        cp = dataclasses.replace(cp, needs_layout_passes=False)
    kern = functools.partial(
        pl.kernel,
        compiler_params=cp,
        out_type=jax.ShapeDtypeStruct((ntok, _DIM), jnp.float32),
        mesh=mesh,
        scratch_types=[
            pltpu.VMEM((tpw, _BAG), jnp.int32),
            pltpu.VMEM((tpw, _BAG), jnp.float32),
            pltpu.VMEM((_GCH, _DIM), jnp.float32),
            pltpu.VMEM((_GCH, _DIM), jnp.float32),
            pltpu.VMEM((_DIM,), jnp.float32),
            pltpu.SemaphoreType.DMA,
            pltpu.SemaphoreType.DMA,
        ],
    )(functools.partial(_bag_body, tpw))
    return kern(values, idx, w)


_BT = 256
_NJ = [32 // (i + 1) for i in range(_KNN)]
_NCAND = sum(_NJ)


def _extract_topk(msk_ref, src, pay, k_out):
    rows, n = src.shape
    sl = (slice(None), slice(0, n))
    msk_ref[sl] = src
    iota = jax.lax.broadcasted_iota(jnp.int32, (rows, k_out), 1)
    big = jnp.int32(1 << 30)

    def body(k, carry):
        vout, pout = carry
        cur = msk_ref[sl]
        m = jnp.max(cur, axis=1, keepdims=True)
        match = cur == m
        psel = jnp.min(jnp.where(match, pay, big), axis=1, keepdims=True)
        msk_ref[sl] = jnp.where(match & (pay == psel), -jnp.inf, cur)
        vout = jnp.where(iota == k, m, vout)
        pout = jnp.where(iota == k, psel, pout)
        return (vout, pout)

    vout0 = jnp.zeros((rows, k_out), jnp.float32)
    pout0 = jnp.zeros((rows, k_out), jnp.int32)
    vout, pout = jax.lax.fori_loop(0, k_out, body, (vout0, pout0))
    return vout, pout


def _k1_body(x_ref, wqt_ref, bq_ref, kt_ref, wst_ref, bs_ref,
             w_ref, idx_ref, gate_ref, msk_ref):
    xb = x_ref[...]
    q = jnp.dot(xb, wqt_ref[...], preferred_element_type=jnp.float32) + bq_ref[...]

    lane9 = jax.lax.broadcasted_iota(jnp.int32, (_BT, _NKEYS), 1)
    w_heads, idx_heads = [], []
    for h in range(_HEADS):
        q1 = q[:, h * _KDIM:h * _KDIM + _HALF]
        q2 = q[:, h * _KDIM + _HALF:(h + 1) * _KDIM]
        s1 = jnp.dot(q1, kt_ref[h, 0], preferred_element_type=jnp.float32)
        s2 = jnp.dot(q2, kt_ref[h, 1], preferred_element_type=jnp.float32)
        v1, i1 = _extract_topk(msk_ref, s1, lane9, _KNN)
        v2, i2 = _extract_topk(msk_ref, s2, lane9, _KNN)

        pieces, codes = [], []
        for i in range(_KNN):
            n = _NJ[i]
            pieces.append(v1[:, i:i + 1] + v2[:, :n])
            codes.append(i * _KNN +
                         jax.lax.broadcasted_iota(jnp.int32, (_BT, n), 1))
        cand = jnp.concatenate(pieces, axis=1)
        code_row = jnp.concatenate(codes, axis=1)
        bv, code = _extract_topk(msk_ref, cand, code_row, _KNN)
        k1 = code >> 5
        k2 = code & 31

        m0 = bv[:, 0:1]
        e = jnp.exp(bv - m0)
        wgt = e / jnp.sum(e, axis=1, keepdims=True)

        i1sel = jnp.zeros((_BT, _KNN), jnp.int32)
        i2sel = jnp.zeros((_BT, _KNN), jnp.int32)
        for j in range(_KNN):
            i1sel = jnp.where(k1 == j, i1[:, j:j + 1], i1sel)
            i2sel = jnp.where(k2 == j, i2[:, j:j + 1], i2sel)
        w_heads.append(wgt)
        idx_heads.append(i1sel * _NKEYS + i2sel)

    w_ref[...] = jnp.concatenate(w_heads, axis=1)
    idx_ref[...] = jnp.concatenate(idx_heads, axis=1)

    z = jnp.dot(xb, wst_ref[...], preferred_element_type=jnp.float32) + bs_ref[...]
    gate_ref[...] = z / (1.0 + jnp.exp(-z))


def _tc_search_and_gate(x, W_qT, b_q2, keysT, W_swiluT, b_s2):
    ntok = x.shape[0]
    grid = (ntok // _BT,)
    return pl.pallas_call(
        _k1_body,
        grid=grid,
        in_specs=[
            pl.BlockSpec((_BT, _DIM), lambda i: (i, 0)),
            pl.BlockSpec((_DIM, _HEADS * _KDIM), lambda i: (0, 0)),
            pl.BlockSpec((1, _HEADS * _KDIM), lambda i: (0, 0)),
            pl.BlockSpec((_HEADS, 2, _HALF, _NKEYS), lambda i: (0, 0, 0, 0)),
            pl.BlockSpec((_DIM, _DIM), lambda i: (0, 0)),
            pl.BlockSpec((1, _DIM), lambda i: (0, 0)),
        ],
        out_specs=[
            pl.BlockSpec((_BT, _BAG), lambda i: (i, 0)),
            pl.BlockSpec((_BT, _BAG), lambda i: (i, 0)),
            pl.BlockSpec((_BT, _DIM), lambda i: (i, 0)),
        ],
        out_shape=[
            jax.ShapeDtypeStruct((ntok, _BAG), jnp.float32),
            jax.ShapeDtypeStruct((ntok, _BAG), jnp.int32),
            jax.ShapeDtypeStruct((ntok, _DIM), jnp.float32),
        ],
        scratch_shapes=[pltpu.VMEM((_BT, _NKEYS), jnp.float32)],
    )(x, W_qT, b_q2, keysT, W_swiluT, b_s2)


def _k3_body(bag_ref, gate_ref, wvt_ref, bv_ref, o_ref):
    o_ref[...] = jnp.dot(bag_ref[...] * gate_ref[...], wvt_ref[...],
                         preferred_element_type=jnp.float32) + bv_ref[...]


def _tc_vproj(bag, gate, W_vprojT, b_v2):
    return pl.pallas_call(
        _k3_body,
        grid=(bag.shape[0] // _BT,),
        in_specs=[
            pl.BlockSpec((_BT, _DIM), lambda i: (i, 0)),
            pl.BlockSpec((_BT, _DIM), lambda i: (i, 0)),
            pl.BlockSpec((_DIM, _DIM), lambda i: (0, 0)),
            pl.BlockSpec((1, _DIM), lambda i: (0, 0)),
        ],
        out_specs=pl.BlockSpec((_BT, _DIM), lambda i: (i, 0)),
        out_shape=jax.ShapeDtypeStruct((bag.shape[0], _DIM), jnp.float32),
    )(bag, gate, W_vprojT, b_v2)


_NCHUNK = 8


def kernel(x, W_q, b_q, keys_p, values, W_swilu, b_swilu, W_vproj, b_vproj):
    keysT = keys_p.reshape(_HEADS, 2, _NKEYS, _HALF).transpose(0, 1, 3, 2)
    W_qT, W_swiluT, W_vprojT = W_q.T, W_swilu.T, W_vproj.T
    bq2, bs2, bv2 = b_q[None, :], b_swilu[None, :], b_vproj[None, :]
    step = _NTOK // _NCHUNK
    outs = []
    for c in range(_NCHUNK):
        xc = jax.lax.slice_in_dim(x, c * step, (c + 1) * step, axis=0)
        w2d, idx2d, gate = _tc_search_and_gate(
            xc, W_qT, bq2, keysT, W_swiluT, bs2)
        bag = _sc_bag(values, idx2d, w2d)
        outs.append(_tc_vproj(bag, gate, W_vprojT, bv2))
    return jnp.concatenate(outs, axis=0)
